# Initial kernel scaffold; baseline (speedup 1.0000x reference)
import jax
import jax.numpy as jnp
from jax.experimental import pallas as pl


def kernel(x, edge_index, W1, b1, W2, b2):
    raise NotImplementedError("write your pallas kernel here")


# speedup vs baseline: 8.0166x; 8.0166x over previous
import functools

import jax
import jax.numpy as jnp
from jax import lax
from jax.experimental import pallas as pl
from jax.experimental.pallas import tpu as pltpu
from jax.experimental.pallas import tpu_sc as plsc

N_NODES = 10000
N_EDGES = 160000
D_FEAT = 256
QW = 64

NC = 2
NS = 16

EDGES_PER_TILE = N_EDGES // NS
ROWS_PER_TILE = 632
N_PAD = NS * ROWS_PER_TILE

DEG_CHUNK = 1000
DEG_ITERS = EDGES_PER_TILE // DEG_CHUNK
DEG_W = 16

AGG_CHUNK = 400
AGG_ITERS = EDGES_PER_TILE // AGG_CHUNK

_SC_MESH = plsc.VectorSubcoreMesh(core_axis_name="c", subcore_axis_name="s")
_SC_PARAMS = pltpu.CompilerParams(use_tc_tiling_on_sc=False)


@functools.partial(
    pl.kernel,
    out_type=jax.ShapeDtypeStruct((N_PAD, DEG_W), jnp.float32),
    mesh=_SC_MESH,
    scratch_types=[
        pltpu.VMEM((DEG_CHUNK, DEG_W), jnp.float32),
        pltpu.VMEM((DEG_CHUNK,), jnp.int32),
        pltpu.VMEM_SHARED((N_PAD, DEG_W), jnp.float32),
    ],
    compiler_params=_SC_PARAMS,
)
def _deg_kernel(dst_hbm, zeros_hbm, degx_hbm, ones_v, idx_v, acc):
    c = lax.axis_index("c")
    s = lax.axis_index("s")

    def fill(i, _):
        ones_v[i, :] = jnp.ones((DEG_W,), jnp.float32)
        return 0
    lax.fori_loop(0, DEG_CHUNK, fill, 0)

    row0 = s * ROWS_PER_TILE
    pltpu.sync_copy(zeros_hbm.at[pl.ds(row0, ROWS_PER_TILE)],
                    acc.at[pl.ds(row0, ROWS_PER_TILE)])
    plsc.subcore_barrier()

    def body(i, _):
        base = s * EDGES_PER_TILE + i * DEG_CHUNK
        pltpu.sync_copy(dst_hbm.at[pl.ds(base, DEG_CHUNK)], idx_v)
        pltpu.sync_copy(ones_v, acc.at[idx_v], add=True)
        return 0
    lax.fori_loop(0, DEG_ITERS, body, 0)
    plsc.subcore_barrier()

    @pl.when(c == 0)
    def _():
        pltpu.sync_copy(acc.at[pl.ds(row0, ROWS_PER_TILE)],
                        degx_hbm.at[pl.ds(row0, ROWS_PER_TILE)])


@functools.partial(
    pl.kernel,
    out_type=tuple(
        jax.ShapeDtypeStruct((N_PAD, QW), jnp.float32) for _ in range(4)
    ),
    mesh=_SC_MESH,
    scratch_types=[
        pltpu.VMEM((AGG_CHUNK,), jnp.int32),
        pltpu.VMEM((AGG_CHUNK,), jnp.int32),
        pltpu.VMEM((AGG_CHUNK, QW), jnp.float32),
        pltpu.SemaphoreType.DMA,
        pltpu.VMEM_SHARED((N_PAD, QW), jnp.float32),
    ],
    compiler_params=_SC_PARAMS,
)
def _agg_kernel(g0_hbm, g1_hbm, g2_hbm, g3_hbm, src_hbm, dst_hbm, zeros_hbm,
                o0_hbm, o1_hbm, o2_hbm, o3_hbm, idx_s, idx_d, rows, sem, acc):
    c = lax.axis_index("c")
    s = lax.axis_index("s")
    row0 = s * ROWS_PER_TILE

    def run_quarter(tab_hbm, out_hbm):
        pltpu.sync_copy(zeros_hbm.at[pl.ds(row0, ROWS_PER_TILE)],
                        acc.at[pl.ds(row0, ROWS_PER_TILE)])
        plsc.subcore_barrier()

        def body(i, _):
            base = s * EDGES_PER_TILE + i * AGG_CHUNK
            pltpu.sync_copy(src_hbm.at[pl.ds(base, AGG_CHUNK)], idx_s)
            pltpu.sync_copy(dst_hbm.at[pl.ds(base, AGG_CHUNK)], idx_d)
            pltpu.async_copy(tab_hbm.at[idx_s], rows, sem).wait()
            pltpu.sync_copy(rows, acc.at[idx_d], add=True)
            return 0
        lax.fori_loop(0, AGG_ITERS, body, 0)
        plsc.subcore_barrier()
        pltpu.sync_copy(acc.at[pl.ds(row0, ROWS_PER_TILE)],
                        out_hbm.at[pl.ds(row0, ROWS_PER_TILE)])

    @pl.when(c == 0)
    def _():
        run_quarter(g0_hbm, o0_hbm)
        run_quarter(g1_hbm, o1_hbm)

    @pl.when(c == 1)
    def _():
        run_quarter(g2_hbm, o2_hbm)
        run_quarter(g3_hbm, o3_hbm)


BR = 1000
GRID = N_NODES // BR


def _r_from_degx(degx):
    deg = degx[:, 0:1]
    return lax.rsqrt(jnp.maximum(deg, 1.0))


def _store_quarters(g, refs):
    for q, ref in enumerate(refs):
        ref[...] = g[:, q * QW:(q + 1) * QW]


def _enc1_body(x_ref, w1_ref, b1_ref, degx_ref, *g_refs):
    r = _r_from_degx(degx_ref[...])
    h = jnp.dot(x_ref[...], w1_ref[...], preferred_element_type=jnp.float32)
    g = (h + b1_ref[...][None, :]) * r
    _store_quarters(g, g_refs)


def _enc2_body(a0, a1, a2, a3, degx_ref, w2_ref, b2_ref, *g_refs):
    r = _r_from_degx(degx_ref[...])
    agg = jnp.concatenate([a0[...], a1[...], a2[...], a3[...]], axis=1)
    h1 = jnp.maximum(agg * r, 0.0)
    h2 = jnp.dot(h1, w2_ref[...], preferred_element_type=jnp.float32)
    g = (h2 + b2_ref[...][None, :]) * r
    _store_quarters(g, g_refs)


def _final_body(a0, a1, a2, a3, degx_ref, out_ref):
    r = _r_from_degx(degx_ref[...])
    agg = jnp.concatenate([a0[...], a1[...], a2[...], a3[...]], axis=1)
    out_ref[...] = agg * r


def _row_spec(w):
    return pl.BlockSpec((BR, w), lambda i: (i, 0))


def _full_spec(shape):
    return pl.BlockSpec(shape, lambda i: tuple(0 for _ in shape))


_QUARTER_OUT = [
    jax.ShapeDtypeStruct((N_NODES, QW), jnp.float32) for _ in range(4)
]


def _enc1(x, W1, b1, degx):
    return pl.pallas_call(
        _enc1_body,
        grid=(GRID,),
        in_specs=[
            _row_spec(D_FEAT),
            _full_spec((D_FEAT, D_FEAT)),
            _full_spec((D_FEAT,)),
            _row_spec(DEG_W),
        ],
        out_specs=[_row_spec(QW)] * 4,
        out_shape=_QUARTER_OUT,
    )(x, W1, b1, degx)


def _enc2(aggs, degx, W2, b2):
    return pl.pallas_call(
        _enc2_body,
        grid=(GRID,),
        in_specs=[_row_spec(QW)] * 4 + [
            _row_spec(DEG_W),
            _full_spec((D_FEAT, D_FEAT)),
            _full_spec((D_FEAT,)),
        ],
        out_specs=[_row_spec(QW)] * 4,
        out_shape=_QUARTER_OUT,
    )(*aggs, degx, W2, b2)


def _final(aggs, degx):
    return pl.pallas_call(
        _final_body,
        grid=(GRID,),
        in_specs=[_row_spec(QW)] * 4 + [_row_spec(DEG_W)],
        out_specs=_row_spec(D_FEAT),
        out_shape=jax.ShapeDtypeStruct((N_NODES, D_FEAT), jnp.float32),
    )(*aggs, degx)


def kernel(x, edge_index, W1, b1, W2, b2):
    src = edge_index[0]
    dst = edge_index[1]
    zeros_deg = jnp.zeros((N_PAD, DEG_W), jnp.float32)
    zeros_q = jnp.zeros((N_PAD, QW), jnp.float32)

    degx = _deg_kernel(dst, zeros_deg)
    g1 = _enc1(x, W1, b1, degx)
    a1 = _agg_kernel(*g1, src, dst, zeros_q)
    g2 = _enc2(a1, degx, W2, b2)
    a2 = _agg_kernel(*g2, src, dst, zeros_q)
    return _final(a2, degx)


# speedup vs baseline: 11.3876x; 1.4205x over previous
import functools

import jax
import jax.numpy as jnp
from jax import lax
from jax.experimental import pallas as pl
from jax.experimental.pallas import tpu as pltpu
from jax.experimental.pallas import tpu_sc as plsc

N_NODES = 10000
N_EDGES = 160000
D_FEAT = 256
QW = 64

NC = 2
NS = 16

EDGES_PER_TILE = N_EDGES // NS
ROWS_PER_TILE = 632
N_PAD = NS * ROWS_PER_TILE

DEG_CHUNK = 1000
DEG_ITERS = EDGES_PER_TILE // DEG_CHUNK
DEG_W = 16

AGG_CHUNK = 400
AGG_ITERS = EDGES_PER_TILE // AGG_CHUNK

_SC_MESH = plsc.VectorSubcoreMesh(core_axis_name="c", subcore_axis_name="s")
_SC_PARAMS = pltpu.CompilerParams(use_tc_tiling_on_sc=False)


@functools.partial(
    pl.kernel,
    out_type=jax.ShapeDtypeStruct((N_PAD, DEG_W), jnp.float32),
    mesh=_SC_MESH,
    scratch_types=[
        pltpu.VMEM((DEG_CHUNK, DEG_W), jnp.float32),
        pltpu.VMEM((DEG_CHUNK,), jnp.int32),
        pltpu.VMEM_SHARED((N_PAD, DEG_W), jnp.float32),
    ],
    compiler_params=_SC_PARAMS,
)
def _deg_kernel(dst_hbm, zeros_hbm, degx_hbm, ones_v, idx_v, acc):
    c = lax.axis_index("c")
    s = lax.axis_index("s")

    def fill(i, _):
        ones_v[i, :] = jnp.ones((DEG_W,), jnp.float32)
        return 0
    lax.fori_loop(0, DEG_CHUNK, fill, 0)

    row0 = s * ROWS_PER_TILE
    pltpu.sync_copy(zeros_hbm.at[pl.ds(row0, ROWS_PER_TILE)],
                    acc.at[pl.ds(row0, ROWS_PER_TILE)])
    plsc.subcore_barrier()

    def body(i, _):
        base = s * EDGES_PER_TILE + i * DEG_CHUNK
        pltpu.sync_copy(dst_hbm.at[pl.ds(base, DEG_CHUNK)], idx_v)
        pltpu.sync_copy(ones_v, acc.at[idx_v], add=True)
        return 0
    lax.fori_loop(0, DEG_ITERS, body, 0)
    plsc.subcore_barrier()

    @pl.when(c == 0)
    def _():
        pltpu.sync_copy(acc.at[pl.ds(row0, ROWS_PER_TILE)],
                        degx_hbm.at[pl.ds(row0, ROWS_PER_TILE)])


@functools.partial(
    pl.kernel,
    out_type=tuple(
        jax.ShapeDtypeStruct((N_PAD, QW), jnp.float32) for _ in range(4)
    ),
    mesh=_SC_MESH,
    scratch_types=[
        pltpu.VMEM((AGG_ITERS, AGG_CHUNK), jnp.int32),
        pltpu.VMEM((AGG_ITERS, AGG_CHUNK), jnp.int32),
        pltpu.VMEM((AGG_CHUNK, QW), jnp.float32),
        pltpu.VMEM((AGG_CHUNK, QW), jnp.float32),
        pltpu.SemaphoreType.DMA,
        pltpu.SemaphoreType.DMA,
        pltpu.SemaphoreType.DMA,
        pltpu.SemaphoreType.DMA,
        pltpu.VMEM_SHARED((N_PAD, QW), jnp.float32),
    ],
    compiler_params=_SC_PARAMS,
)
def _agg_kernel(g0_hbm, g1_hbm, g2_hbm, g3_hbm, src_hbm, dst_hbm, zeros_hbm,
                o0_hbm, o1_hbm, o2_hbm, o3_hbm, sidx, didx,
                rows0, rows1, gs0, gs1, ss0, ss1, acc):
    c = lax.axis_index("c")
    s = lax.axis_index("s")
    row0 = s * ROWS_PER_TILE
    rows = (rows0, rows1)
    gsem = (gs0, gs1)
    ssem = (ss0, ss1)

    pltpu.sync_copy(src_hbm.at[s], sidx)
    pltpu.sync_copy(dst_hbm.at[s], didx)

    def run_quarter(tab_hbm, out_hbm):
        pltpu.sync_copy(zeros_hbm.at[pl.ds(row0, ROWS_PER_TILE)],
                        acc.at[pl.ds(row0, ROWS_PER_TILE)])
        plsc.subcore_barrier()

        def gather(i):
            b = i % 2
            return pltpu.async_copy(tab_hbm.at[sidx.at[i]], rows[b], gsem[b])

        g_descs = [None] * AGG_ITERS
        s_descs = [None] * AGG_ITERS
        g_descs[0] = gather(0)
        for i in range(AGG_ITERS):
            b = i % 2
            g_descs[i].wait()
            s_descs[i] = pltpu.async_copy(rows[b], acc.at[didx.at[i]],
                                          ssem[b], add=True)
            if i + 1 < AGG_ITERS:
                if i >= 1:
                    s_descs[i - 1].wait()
                g_descs[i + 1] = gather(i + 1)
        s_descs[AGG_ITERS - 1].wait()
        if AGG_ITERS >= 2:
            s_descs[AGG_ITERS - 2].wait()
        plsc.subcore_barrier()
        pltpu.sync_copy(acc.at[pl.ds(row0, ROWS_PER_TILE)],
                        out_hbm.at[pl.ds(row0, ROWS_PER_TILE)])

    @pl.when(c == 0)
    def _():
        run_quarter(g0_hbm, o0_hbm)
        run_quarter(g1_hbm, o1_hbm)

    @pl.when(c == 1)
    def _():
        run_quarter(g2_hbm, o2_hbm)
        run_quarter(g3_hbm, o3_hbm)


BR = 1000
GRID = N_NODES // BR


def _r_from_degx(degx):
    deg = degx[:, 0:1]
    return lax.rsqrt(jnp.maximum(deg, 1.0))


def _store_quarters(g, refs):
    for q, ref in enumerate(refs):
        ref[...] = g[:, q * QW:(q + 1) * QW]


def _enc1_body(x_ref, w1_ref, b1_ref, degx_ref, *g_refs):
    r = _r_from_degx(degx_ref[...])
    h = jnp.dot(x_ref[...], w1_ref[...], preferred_element_type=jnp.float32)
    g = (h + b1_ref[...][None, :]) * r
    _store_quarters(g, g_refs)


def _enc2_body(a0, a1, a2, a3, degx_ref, w2_ref, b2_ref, *g_refs):
    r = _r_from_degx(degx_ref[...])
    agg = jnp.concatenate([a0[...], a1[...], a2[...], a3[...]], axis=1)
    h1 = jnp.maximum(agg * r, 0.0)
    h2 = jnp.dot(h1, w2_ref[...], preferred_element_type=jnp.float32)
    g = (h2 + b2_ref[...][None, :]) * r
    _store_quarters(g, g_refs)


def _final_body(a0, a1, a2, a3, degx_ref, out_ref):
    r = _r_from_degx(degx_ref[...])
    agg = jnp.concatenate([a0[...], a1[...], a2[...], a3[...]], axis=1)
    out_ref[...] = agg * r


def _row_spec(w):
    return pl.BlockSpec((BR, w), lambda i: (i, 0))


def _full_spec(shape):
    return pl.BlockSpec(shape, lambda i: tuple(0 for _ in shape))


_QUARTER_OUT = [
    jax.ShapeDtypeStruct((N_NODES, QW), jnp.float32) for _ in range(4)
]


def _enc1(x, W1, b1, degx):
    return pl.pallas_call(
        _enc1_body,
        grid=(GRID,),
        in_specs=[
            _row_spec(D_FEAT),
            _full_spec((D_FEAT, D_FEAT)),
            _full_spec((D_FEAT,)),
            _row_spec(DEG_W),
        ],
        out_specs=[_row_spec(QW)] * 4,
        out_shape=_QUARTER_OUT,
    )(x, W1, b1, degx)


def _enc2(aggs, degx, W2, b2):
    return pl.pallas_call(
        _enc2_body,
        grid=(GRID,),
        in_specs=[_row_spec(QW)] * 4 + [
            _row_spec(DEG_W),
            _full_spec((D_FEAT, D_FEAT)),
            _full_spec((D_FEAT,)),
        ],
        out_specs=[_row_spec(QW)] * 4,
        out_shape=_QUARTER_OUT,
    )(*aggs, degx, W2, b2)


def _final(aggs, degx):
    return pl.pallas_call(
        _final_body,
        grid=(GRID,),
        in_specs=[_row_spec(QW)] * 4 + [_row_spec(DEG_W)],
        out_specs=_row_spec(D_FEAT),
        out_shape=jax.ShapeDtypeStruct((N_NODES, D_FEAT), jnp.float32),
    )(*aggs, degx)


def kernel(x, edge_index, W1, b1, W2, b2):
    src = edge_index[0]
    dst = edge_index[1]
    src3 = src.reshape(NS, AGG_ITERS, AGG_CHUNK)
    dst3 = dst.reshape(NS, AGG_ITERS, AGG_CHUNK)
    zeros_deg = jnp.zeros((N_PAD, DEG_W), jnp.float32)
    zeros_q = jnp.zeros((N_PAD, QW), jnp.float32)

    degx = _deg_kernel(dst, zeros_deg)
    g1 = _enc1(x, W1, b1, degx)
    a1 = _agg_kernel(*g1, src3, dst3, zeros_q)
    g2 = _enc2(a1, degx, W2, b2)
    a2 = _agg_kernel(*g2, src3, dst3, zeros_q)
    return _final(a2, degx)


# speedup vs baseline: 14.8525x; 1.3043x over previous
import functools

import jax
import jax.numpy as jnp
from jax import lax
from jax.experimental import pallas as pl
from jax.experimental.pallas import tpu as pltpu
from jax.experimental.pallas import tpu_sc as plsc

N_NODES = 10000
N_EDGES = 160000
D_FEAT = 256
QW = 64

NC = 2
NS = 16

EDGES_PER_TILE = N_EDGES // NS
ROWS_PER_TILE = 632
N_PAD = NS * ROWS_PER_TILE

DEG_CHUNK = 2000
DEG_ITERS = EDGES_PER_TILE // DEG_CHUNK
DEG_W = 16

AGG_CHUNK = 400
AGG_ITERS = EDGES_PER_TILE // AGG_CHUNK

_SC_MESH = plsc.VectorSubcoreMesh(core_axis_name="c", subcore_axis_name="s")
_SC_PARAMS = pltpu.CompilerParams(use_tc_tiling_on_sc=False)


@functools.partial(
    pl.kernel,
    out_type=jax.ShapeDtypeStruct((N_PAD, DEG_W), jnp.float32),
    mesh=_SC_MESH,
    scratch_types=[
        pltpu.VMEM((DEG_CHUNK, DEG_W), jnp.float32),
        pltpu.VMEM((DEG_CHUNK,), jnp.int32),
        pltpu.VMEM((DEG_CHUNK,), jnp.int32),
        pltpu.SemaphoreType.DMA,
        pltpu.SemaphoreType.DMA,
        pltpu.VMEM_SHARED((N_PAD, DEG_W), jnp.float32),
    ],
    compiler_params=_SC_PARAMS,
)
def _deg_kernel(edges_hbm, ones_hbm, zeros_hbm, degx_hbm,
                ones_v, idx0, idx1, ss0, ss1, acc):
    c = lax.axis_index("c")
    s = lax.axis_index("s")
    idx = (idx0, idx1)
    ssem = (ss0, ss1)

    pltpu.sync_copy(ones_hbm, ones_v)
    row0 = s * ROWS_PER_TILE
    pltpu.sync_copy(zeros_hbm.at[pl.ds(row0, ROWS_PER_TILE)],
                    acc.at[pl.ds(row0, ROWS_PER_TILE)])
    plsc.subcore_barrier()

    s_descs = [None] * DEG_ITERS
    for i in range(DEG_ITERS):
        b = i % 2
        if i >= 2:
            s_descs[i - 2].wait()
        base = s * EDGES_PER_TILE + i * DEG_CHUNK
        pltpu.sync_copy(edges_hbm.at[1, pl.ds(base, DEG_CHUNK)], idx[b])
        s_descs[i] = pltpu.async_copy(ones_v, acc.at[idx[b]], ssem[b],
                                      add=True)
    s_descs[DEG_ITERS - 1].wait()
    if DEG_ITERS >= 2:
        s_descs[DEG_ITERS - 2].wait()
    plsc.subcore_barrier()

    @pl.when(c == 0)
    def _():
        pltpu.sync_copy(acc.at[pl.ds(row0, ROWS_PER_TILE)],
                        degx_hbm.at[pl.ds(row0, ROWS_PER_TILE)])


IDX_VECS = AGG_CHUNK // 16


@functools.partial(
    pl.kernel,
    out_type=(
        jax.ShapeDtypeStruct((N_PAD, 2 * QW), jnp.float32),
        jax.ShapeDtypeStruct((N_PAD, 2 * QW), jnp.float32),
    ),
    mesh=_SC_MESH,
    scratch_types=[
        pltpu.VMEM((AGG_ITERS, AGG_CHUNK), jnp.int32),
        pltpu.VMEM((AGG_ITERS, AGG_CHUNK), jnp.int32),
        pltpu.VMEM((AGG_CHUNK, QW), jnp.float32),
        pltpu.VMEM((AGG_CHUNK, QW), jnp.float32),
        pltpu.SemaphoreType.DMA,
        pltpu.SemaphoreType.DMA,
        pltpu.SemaphoreType.DMA,
        pltpu.SemaphoreType.DMA,
        pltpu.VMEM_SHARED((N_PAD, QW), jnp.float32),
    ],
    compiler_params=_SC_PARAMS,
)
def _agg_kernel(t0_hbm, t1_hbm, edges_hbm, zeros_hbm,
                o0_hbm, o1_hbm, sidx2, didx,
                rows0, rows1, gs0, gs1, ss0, ss1, acc):
    c = lax.axis_index("c")
    s = lax.axis_index("s")
    row0 = s * ROWS_PER_TILE
    rows = (rows0, rows1)
    gsem = (gs0, gs1)
    ssem = (ss0, ss1)

    pltpu.sync_copy(edges_hbm.at[0, s], sidx2)
    pltpu.sync_copy(edges_hbm.at[1, s], didx)

    def xform(mul, add):
        def xform_row(i, _):
            def xform_vec(k, _):
                v = sidx2[i, pl.ds(k * 16, 16)]
                sidx2[i, pl.ds(k * 16, 16)] = v * mul + add
                return 0
            lax.fori_loop(0, IDX_VECS, xform_vec, 0)
            return 0
        lax.fori_loop(0, AGG_ITERS, xform_row, 0)

    def run_pipe(tab_hbm):
        def gather(i):
            b = i % 2
            return pltpu.async_copy(tab_hbm.at[sidx2.at[i]], rows[b], gsem[b])

        g_descs = [None] * AGG_ITERS
        s_descs = [None] * AGG_ITERS
        g_descs[0] = gather(0)
        yield
        for i in range(AGG_ITERS):
            b = i % 2
            g_descs[i].wait()
            s_descs[i] = pltpu.async_copy(rows[b], acc.at[didx.at[i]],
                                          ssem[b], add=True)
            if i + 1 < AGG_ITERS:
                if i >= 1:
                    s_descs[i - 1].wait()
                g_descs[i + 1] = gather(i + 1)
        s_descs[AGG_ITERS - 1].wait()
        if AGG_ITERS >= 2:
            s_descs[AGG_ITERS - 2].wait()

    def writeout(out_hbm, q):
        pltpu.sync_copy(acc.at[pl.ds(row0, ROWS_PER_TILE)],
                        out_hbm.at[pl.ds(row0, ROWS_PER_TILE),
                                   pl.ds(q * QW, QW)])

    def run_core(tab_hbm, out_hbm):
        xform(2, 0)
        pipe = run_pipe(tab_hbm)
        next(pipe)
        pltpu.sync_copy(zeros_hbm.at[pl.ds(row0, ROWS_PER_TILE)],
                        acc.at[pl.ds(row0, ROWS_PER_TILE)])
        plsc.subcore_barrier()
        for _ in pipe:
            pass
        plsc.subcore_barrier()
        writeout(out_hbm, 0)
        xform(1, 1)
        pipe = run_pipe(tab_hbm)
        next(pipe)
        plsc.subcore_barrier()
        for _ in pipe:
            pass
        plsc.subcore_barrier()
        writeout(out_hbm, 1)

    @pl.when(c == 0)
    def _():
        run_core(t0_hbm, o0_hbm)

    @pl.when(c == 1)
    def _():
        run_core(t1_hbm, o1_hbm)


BR = 2000
GRID = N_NODES // BR


def _r_from_degx(degx):
    deg = degx[:, 0:1]
    return lax.rsqrt(jnp.maximum(deg, 1.0))


def _enc1_body(x_ref, w1_ref, b1_ref, degx_ref, p0_ref, p1_ref):
    r = _r_from_degx(degx_ref[...])
    h = jnp.dot(x_ref[...], w1_ref[...], preferred_element_type=jnp.float32)
    g = (h + b1_ref[...][None, :]) * r
    p0_ref[...] = g[:, :2 * QW]
    p1_ref[...] = g[:, 2 * QW:]


def _unmix(a0, a1):
    lo0, hi0 = a0[:, :QW], a0[:, QW:]
    lo1, hi1 = a1[:, :QW], a1[:, QW:]
    return jnp.concatenate([lo0, hi0 - lo0, lo1, hi1 - lo1], axis=1)


def _enc2_body(a0, a1, degx_ref, w2_ref, b2_ref, p0_ref, p1_ref):
    r = _r_from_degx(degx_ref[...])
    agg = _unmix(a0[...], a1[...])
    h1 = jnp.maximum(agg * r, 0.0)
    h2 = jnp.dot(h1, w2_ref[...], preferred_element_type=jnp.float32)
    g = (h2 + b2_ref[...][None, :]) * r
    p0_ref[...] = g[:, :2 * QW]
    p1_ref[...] = g[:, 2 * QW:]


def _final_body(a0, a1, degx_ref, out_ref):
    r = _r_from_degx(degx_ref[...])
    out_ref[...] = _unmix(a0[...], a1[...]) * r


def _row_spec(w):
    return pl.BlockSpec((BR, w), lambda i: (i, 0))


def _full_spec(shape):
    return pl.BlockSpec(shape, lambda i: tuple(0 for _ in shape))


_PACKED_OUT = [
    jax.ShapeDtypeStruct((N_NODES, 2 * QW), jnp.float32) for _ in range(2)
]


def _enc1(x, W1, b1, degx):
    return pl.pallas_call(
        _enc1_body,
        grid=(GRID,),
        in_specs=[
            _row_spec(D_FEAT),
            _full_spec((D_FEAT, D_FEAT)),
            _full_spec((D_FEAT,)),
            _row_spec(DEG_W),
        ],
        out_specs=[_row_spec(2 * QW)] * 2,
        out_shape=_PACKED_OUT,
    )(x, W1, b1, degx)


def _enc2(aggs, degx, W2, b2):
    return pl.pallas_call(
        _enc2_body,
        grid=(GRID,),
        in_specs=[_row_spec(2 * QW)] * 2 + [
            _row_spec(DEG_W),
            _full_spec((D_FEAT, D_FEAT)),
            _full_spec((D_FEAT,)),
        ],
        out_specs=[_row_spec(2 * QW)] * 2,
        out_shape=_PACKED_OUT,
    )(*aggs, degx, W2, b2)


def _final(aggs, degx):
    return pl.pallas_call(
        _final_body,
        grid=(GRID,),
        in_specs=[_row_spec(2 * QW)] * 2 + [_row_spec(DEG_W)],
        out_specs=_row_spec(D_FEAT),
        out_shape=jax.ShapeDtypeStruct((N_NODES, D_FEAT), jnp.float32),
    )(*aggs, degx)


def kernel(x, edge_index, W1, b1, W2, b2):
    edges4 = edge_index.reshape(2, NS, AGG_ITERS, AGG_CHUNK)
    ones_deg = jnp.ones((DEG_CHUNK, DEG_W), jnp.float32)
    zeros_deg = jnp.zeros((N_PAD, DEG_W), jnp.float32)
    zeros_q = jnp.zeros((N_PAD, QW), jnp.float32)

    def tables(gp):
        return [g.reshape(2 * N_NODES, QW) for g in gp]

    degx = _deg_kernel(edge_index, ones_deg, zeros_deg)
    g1 = _enc1(x, W1, b1, degx)
    a1 = _agg_kernel(*tables(g1), edges4, zeros_q)
    g2 = _enc2(a1, degx, W2, b2)
    a2 = _agg_kernel(*tables(g2), edges4, zeros_q)
    return _final(a2, degx)
